# baseline (speedup 1.0000x reference)
import functools

import jax
import jax.numpy as jnp
from jax import lax
from jax.experimental import pallas as pl
from jax.experimental.pallas import tpu as pltpu
from jax.experimental.pallas import tpu_sc as plsc

_BATCH = 4096
_DIM = 64
_L = 16

_NC = 2
_NS = 16
_NW = _NC * _NS
_BPW = _BATCH // _NW

_mesh = plsc.VectorSubcoreMesh(core_axis_name="c", subcore_axis_name="s")


@functools.partial(
    pl.kernel,
    mesh=_mesh,
    out_type=jax.ShapeDtypeStruct((_BATCH, _DIM), jnp.float32),
    scratch_types=[
        pltpu.VMEM((_BPW,), jnp.int32),
        pltpu.VMEM((_BPW, _DIM), jnp.float32),
        pltpu.VMEM((_BPW, _DIM), jnp.float32),
        pltpu.SemaphoreType.DMA,
        pltpu.SemaphoreType.DMA,
    ],
    compiler_params=pltpu.CompilerParams(use_tc_tiling_on_sc=False),
)
def _gather_add(x_hbm, const_hbm, idx_hbm, out_hbm, idx_v, rows_v, x_v,
                sem_g, sem_x):
    wid = lax.axis_index("s") * _NC + lax.axis_index("c")
    base = wid * _BPW
    pltpu.sync_copy(idx_hbm.at[pl.ds(base, _BPW)], idx_v)
    cp_x = pltpu.async_copy(x_hbm.at[pl.ds(base, _BPW)], x_v, sem_x)
    cp_g = pltpu.async_copy(const_hbm.at[idx_v], rows_v, sem_g)
    cp_g.wait()
    cp_x.wait()

    def row(i, carry):
        for j in range(_DIM // _L):
            sl = pl.ds(j * _L, _L)
            rows_v[i, sl] = rows_v[i, sl] + x_v[i, sl]
        return carry

    lax.fori_loop(0, _BPW, row, 0)
    pltpu.sync_copy(rows_v, out_hbm.at[pl.ds(base, _BPW)])


def kernel(x, const, indices):
    return _gather_add(x, const, indices.astype(jnp.int32))
